# baseline (speedup 1.0000x reference)
import functools

import jax
import jax.numpy as jnp
from jax import lax
from jax.experimental import pallas as pl
from jax.experimental.pallas import tpu as pltpu
from jax.experimental.pallas import tpu_sc as plsc

_G = 128
_IK = 8


@functools.lru_cache(maxsize=None)
def _make_gather(n_groups, v_rows, d):
    info = plsc.get_sparse_core_info()
    nc, ns = info.num_cores, info.num_subcores
    nw = nc * ns
    g_per_w = n_groups // nw
    n_chunks = g_per_w // _IK

    mesh = plsc.VectorSubcoreMesh(core_axis_name="c", subcore_axis_name="s")

    @functools.partial(
        pl.kernel,
        mesh=mesh,
        compiler_params=pltpu.CompilerParams(use_tc_tiling_on_sc=False),
        out_type=jax.ShapeDtypeStruct((n_groups, _G, d), jnp.float32),
        scratch_types=[
            pltpu.VMEM((_IK, _G), jnp.int32),
            pltpu.VMEM((_IK, _G, d), jnp.float32),
            pltpu.SemaphoreType.DMA,
        ],
    )
    def gather(table_hbm, idx_hbm, out_hbm, idx_v, rows_v, sem):
        wid = lax.axis_index("s") * nc + lax.axis_index("c")
        gbase = wid * g_per_w

        def body(ch, carry):
            roff = gbase + ch * _IK
            pltpu.sync_copy(idx_hbm.at[pl.ds(roff, _IK)], idx_v)
            copies = [
                pltpu.async_copy(table_hbm.at[idx_v.at[j]], rows_v.at[j], sem)
                for j in range(_IK)
            ]
            for cp in copies:
                cp.wait()
            pltpu.sync_copy(rows_v, out_hbm.at[pl.ds(roff, _IK)])
            return carry

        lax.fori_loop(0, n_chunks, body, 0)

    return gather


def kernel(x, table):
    b, s = x.shape
    v, d = table.shape
    n = b * s
    idx = x.reshape(n // _G, _G)
    out = _make_gather(n // _G, v, d)(table, idx)
    return out.reshape(b, s, d)


# speedup vs baseline: 1.0186x; 1.0186x over previous
import functools

import jax
import jax.numpy as jnp
from jax import lax
from jax.experimental import pallas as pl
from jax.experimental.pallas import tpu as pltpu
from jax.experimental.pallas import tpu_sc as plsc

_G = 128
_IK = 5


@functools.lru_cache(maxsize=None)
def _make_gather(n_groups, v_rows, d):
    info = plsc.get_sparse_core_info()
    nc, ns = info.num_cores, info.num_subcores
    nw = nc * ns
    g_per_w = n_groups // nw
    n_chunks = g_per_w // _IK
    assert n_chunks >= 4 and n_chunks % 2 == 0

    mesh = plsc.VectorSubcoreMesh(core_axis_name="c", subcore_axis_name="s")

    @functools.partial(
        pl.kernel,
        mesh=mesh,
        compiler_params=pltpu.CompilerParams(use_tc_tiling_on_sc=False),
        out_type=jax.ShapeDtypeStruct((n_groups, _G, d), jnp.float32),
        scratch_types=[
            pltpu.VMEM((g_per_w, _G), jnp.int32),
            pltpu.VMEM((2, _IK, _G, d), jnp.float32),
            pltpu.SemaphoreType.DMA((2,)),
            pltpu.SemaphoreType.DMA((2,)),
        ],
    )
    def gather(table_hbm, idx_hbm, out_hbm, idx_v, rows_v, gsem, ssem):
        wid = lax.axis_index("s") * nc + lax.axis_index("c")
        gbase = wid * g_per_w

        pltpu.sync_copy(idx_hbm.at[pl.ds(gbase, g_per_w)], idx_v)

        def fire_gathers(ch, b):
            for j in range(_IK):
                pltpu.async_copy(
                    table_hbm.at[idx_v.at[ch * _IK + j]],
                    rows_v.at[b, j],
                    gsem.at[b],
                )

        def drain_gathers(ch, b):
            for j in range(_IK):
                pltpu.make_async_copy(
                    table_hbm.at[idx_v.at[ch * _IK + j]],
                    rows_v.at[b, j],
                    gsem.at[b],
                ).wait()

        def fire_store(ch, b):
            return pltpu.async_copy(
                rows_v.at[b], out_hbm.at[pl.ds(gbase + ch * _IK, _IK)],
                ssem.at[b],
            )

        def wait_store(ch, b):
            pltpu.make_async_copy(
                rows_v.at[b], out_hbm.at[pl.ds(gbase + ch * _IK, _IK)],
                ssem.at[b],
            ).wait()

        fire_gathers(0, 0)
        fire_gathers(1, 1)
        drain_gathers(0, 0)
        fire_store(0, 0)

        def step(g, b):
            nb = 1 - b
            wait_store(g - 1, nb)
            fire_gathers(g + 1, nb)
            drain_gathers(g, b)
            fire_store(g, b)

        def body(k, carry):
            step(2 * k + 1, 1)
            step(2 * k + 2, 0)
            return carry

        lax.fori_loop(0, (n_chunks - 2) // 2, body, 0)

        drain_gathers(n_chunks - 1, 1)
        fire_store(n_chunks - 1, 1)
        wait_store(n_chunks - 2, 0)
        wait_store(n_chunks - 1, 1)

    return gather


def kernel(x, table):
    b, s = x.shape
    v, d = table.shape
    n = b * s
    idx = x.reshape(n // _G, _G)
    out = _make_gather(n // _G, v, d)(table, idx)
    return out.reshape(b, s, d)
